# Initial kernel scaffold; baseline (speedup 1.0000x reference)
import jax
import jax.numpy as jnp
from jax.experimental import pallas as pl


def kernel(x, edge_index, batch, W1, b1, W2, b2, fc_w, fc_b):
    raise NotImplementedError("write your pallas kernel here")


# speedup vs baseline: 8.1550x; 8.1550x over previous
import functools

import jax
import jax.numpy as jnp
from jax import lax
from jax.experimental import pallas as pl
from jax.experimental.pallas import tpu as pltpu
from jax.experimental.pallas import tpu_sc as plsc

_N = 10000
_E = 320000
_DIN = 128
_DH = 256
_G = 64

_NC = 2
_NS = 16
_CH = 128
_EPAD = 323584
_EPT = _EPAD // _NS
_EPT2 = _EPAD // (_NC * _NS)
_RPT = 624
_RLAST = _N - 15 * _RPT
_TRASH = _N
_NACC = _N + 8

_BLK = 1000

_f32 = jnp.float32


def _sc_mesh():
    return plsc.VectorSubcoreMesh(
        core_axis_name="c", subcore_axis_name="s",
        num_cores=_NC, num_subcores=_NS)


def _zero_acc(zeros_hbm, acc_sh, s):
    @pl.when(s < 15)
    def _():
        pltpu.sync_copy(zeros_hbm.at[pl.ds(0, _RPT)],
                        acc_sh.at[pl.ds(s * _RPT, _RPT)])

    @pl.when(s == 15)
    def _():
        pltpu.sync_copy(zeros_hbm, acc_sh.at[pl.ds(15 * _RPT, _RLAST)])


def _write_out(acc_sh, out_hbm, c, s):
    @pl.when(s < 15)
    def _():
        pltpu.sync_copy(acc_sh.at[pl.ds(s * _RPT, _RPT)],
                        out_hbm.at[pl.ds(c * _N + s * _RPT, _RPT)])

    @pl.when(s == 15)
    def _():
        pltpu.sync_copy(acc_sh.at[pl.ds(15 * _RPT, _RLAST)],
                        out_hbm.at[pl.ds(c * _N + 15 * _RPT, _RLAST)])


def _deg_body(dst_hbm, ones_hbm, zeros_hbm,
              deg_hbm,
              dst_v, ones_v, acc_sh, sem):
    c = lax.axis_index("c")
    s = lax.axis_index("s")
    _zero_acc(zeros_hbm, acc_sh, s)
    pltpu.sync_copy(ones_hbm, ones_v)
    plsc.subcore_barrier()

    base = (c * _NS + s) * _EPT2

    def step(g, carry):
        off = base + g * _CH
        pltpu.sync_copy(dst_hbm.at[pl.ds(off, _CH)], dst_v)
        pltpu.sync_copy(ones_v, acc_sh.at[dst_v], add=True)
        return carry

    lax.fori_loop(0, _EPT2 // _CH, step, 0)
    plsc.subcore_barrier()
    _write_out(acc_sh, deg_hbm, c, s)


def _deg_call(dst_pad, ones128, zeros128):
    fn = pl.kernel(
        _deg_body,
        out_type=jax.ShapeDtypeStruct((2 * _N, 128), _f32),
        mesh=_sc_mesh(),
        scratch_types=[
            pltpu.VMEM((_CH,), jnp.int32),
            pltpu.VMEM((_CH, 128), _f32),
            pltpu.VMEM_SHARED((_NACC, 128), _f32),
            pltpu.SemaphoreType.DMA,
        ],
    )
    return fn(dst_pad, ones128, zeros128)


def _seg_body(src2_hbm, dst_hbm, y_hbm, zeros_hbm,
              out_hbm,
              src_v, dst_v, rows_v, acc_sh, sem):
    c = lax.axis_index("c")
    s = lax.axis_index("s")
    _zero_acc(zeros_hbm, acc_sh, s)
    plsc.subcore_barrier()

    base = c * _EPAD + s * _EPT
    dbase = s * _EPT

    def step(g, carry):
        pltpu.sync_copy(src2_hbm.at[pl.ds(base + g * _CH, _CH)], src_v)
        pltpu.sync_copy(dst_hbm.at[pl.ds(dbase + g * _CH, _CH)], dst_v)
        pltpu.async_copy(y_hbm.at[src_v], rows_v, sem).wait()
        pltpu.sync_copy(rows_v, acc_sh.at[dst_v], add=True)
        return carry

    lax.fori_loop(0, _EPT // _CH, step, 0)
    plsc.subcore_barrier()
    _write_out(acc_sh, out_hbm, c, s)


def _seg_call(src2, dst_pad, y_s, zeros128):
    fn = pl.kernel(
        _seg_body,
        out_type=jax.ShapeDtypeStruct((2 * _N, 128), _f32),
        mesh=_sc_mesh(),
        scratch_types=[
            pltpu.VMEM((_CH,), jnp.int32),
            pltpu.VMEM((_CH,), jnp.int32),
            pltpu.VMEM((_CH, 128), _f32),
            pltpu.VMEM_SHARED((_NACC, 128), _f32),
            pltpu.SemaphoreType.DMA,
        ],
    )
    return fn(src2, dst_pad, y_s, zeros128)


def _dis_block(d_ref):
    deg = d_ref[...][0, :, 0:1] + d_ref[...][1, :, 0:1] + 1.0
    return lax.rsqrt(deg)


def _halves(st):
    return jnp.concatenate([st[0], st[1]], axis=1)


def _k1_body(x_ref, w1_ref, d_ref, y_ref):
    dis = _dis_block(d_ref)
    xw = jnp.dot(x_ref[...], w1_ref[...], preferred_element_type=_f32)
    y = xw * dis
    y_ref[...] = jnp.stack([y[:, :128], y[:, 128:]])


def _k1_call(x, W1, deg_st):
    grid = (_N // _BLK,)
    return pl.pallas_call(
        _k1_body,
        grid=grid,
        in_specs=[
            pl.BlockSpec((_BLK, _DIN), lambda i: (i, 0)),
            pl.BlockSpec((_DIN, _DH), lambda i: (0, 0)),
            pl.BlockSpec((2, _BLK, 128), lambda i: (0, i, 0)),
        ],
        out_specs=pl.BlockSpec((2, _BLK, 128), lambda i: (0, i, 0)),
        out_shape=jax.ShapeDtypeStruct((2, _N, 128), _f32),
    )(x, W1, deg_st)


def _k2_body(a_ref, y_ref, d_ref, b1_ref, w2_ref, o_ref):
    dis = _dis_block(d_ref)
    su = _halves(a_ref[...] + y_ref[...])
    h = jnp.maximum(su * dis + b1_ref[...][0:1, :], 0.0)
    y2 = jnp.dot(h, w2_ref[...], preferred_element_type=_f32) * dis
    o_ref[...] = jnp.stack([y2[:, :128], y2[:, 128:]])


def _k2_call(a_st, y_st, deg_st, b1r, W2):
    grid = (_N // _BLK,)
    st = pl.BlockSpec((2, _BLK, 128), lambda i: (0, i, 0))
    return pl.pallas_call(
        _k2_body,
        grid=grid,
        in_specs=[
            st, st,
            pl.BlockSpec((2, _BLK, 128), lambda i: (0, i, 0)),
            pl.BlockSpec((8, _DH), lambda i: (0, 0)),
            pl.BlockSpec((_DH, _DH), lambda i: (0, 0)),
        ],
        out_specs=st,
        out_shape=jax.ShapeDtypeStruct((2, _N, 128), _f32),
    )(a_st, y_st, deg_st, b1r, W2)


def _k3_body(a_ref, y_ref, d_ref, b2_ref, bat_ref, sums_ref, cnts_ref):
    i = pl.program_id(0)
    dis = _dis_block(d_ref)
    su = _halves(a_ref[...] + y_ref[...])
    h2 = su * dis + b2_ref[...][0:1, :]
    bb = bat_ref[...][:, 0:1]
    gio = lax.broadcasted_iota(jnp.int32, (_BLK, _G), 1)
    oh = (bb == gio).astype(_f32)
    psum = lax.dot_general(oh, h2, (((0,), (0,)), ((), ())),
                           preferred_element_type=_f32)
    pcnt = jnp.broadcast_to(jnp.sum(oh, axis=0)[:, None], (_G, 128))

    @pl.when(i == 0)
    def _():
        sums_ref[...] = psum
        cnts_ref[...] = pcnt

    @pl.when(i > 0)
    def _():
        sums_ref[...] = sums_ref[...] + psum
        cnts_ref[...] = cnts_ref[...] + pcnt


def _k3_call(a_st, y_st, deg_st, b2r, batch_r):
    grid = (_N // _BLK,)
    st = pl.BlockSpec((2, _BLK, 128), lambda i: (0, i, 0))
    return pl.pallas_call(
        _k3_body,
        grid=grid,
        in_specs=[
            st, st,
            pl.BlockSpec((2, _BLK, 128), lambda i: (0, i, 0)),
            pl.BlockSpec((8, _DH), lambda i: (0, 0)),
            pl.BlockSpec((_BLK, 128), lambda i: (i, 0)),
        ],
        out_specs=[
            pl.BlockSpec((_G, _DH), lambda i: (0, 0)),
            pl.BlockSpec((_G, 128), lambda i: (0, 0)),
        ],
        out_shape=[jax.ShapeDtypeStruct((_G, _DH), _f32),
                   jax.ShapeDtypeStruct((_G, 128), _f32)],
    )(a_st, y_st, deg_st, b2r, batch_r)


def _k4_body(sums_ref, cnts_ref, w_ref, b_ref, out_ref):
    pooled = sums_ref[...] / cnts_ref[...][:, 0:1]
    logits = jnp.dot(pooled, w_ref[...], preferred_element_type=_f32)
    out_ref[...] = jax.nn.sigmoid(logits + b_ref[...][0:1, 0:1])


def _k4_call(sums, cnts, fcw_pad, fcb_r):
    return pl.pallas_call(
        _k4_body,
        grid=(1,),
        in_specs=[
            pl.BlockSpec((_G, _DH), lambda i: (0, 0)),
            pl.BlockSpec((_G, 128), lambda i: (0, 0)),
            pl.BlockSpec((_DH, 128), lambda i: (0, 0)),
            pl.BlockSpec((8, 128), lambda i: (0, 0)),
        ],
        out_specs=pl.BlockSpec((_G, 128), lambda i: (0, 0)),
        out_shape=jax.ShapeDtypeStruct((_G, 128), _f32),
    )(sums, cnts, fcw_pad, fcb_r)


def kernel(x, edge_index, batch, W1, b1, W2, b2, fc_w, fc_b):
    npad = _EPAD - _E
    ei = edge_index.astype(jnp.int32)
    src_pad = jnp.concatenate([ei[0], jnp.zeros((npad,), jnp.int32)])
    dst_pad = jnp.concatenate([ei[1], jnp.full((npad,), _TRASH, jnp.int32)])
    src2 = jnp.concatenate([src_pad, src_pad + _N])

    ones128 = jnp.ones((_CH, 128), _f32)
    zeros128 = jnp.zeros((_RLAST, 128), _f32)

    b1r = jnp.broadcast_to(b1[None, :], (8, _DH))
    b2r = jnp.broadcast_to(b2[None, :], (8, _DH))
    fcw_pad = jnp.pad(fc_w, ((0, 0), (0, 127)))
    fcb_r = jnp.broadcast_to(fc_b[None, :], (8, 128))
    batch_r = jnp.broadcast_to(batch.astype(jnp.int32)[:, None], (_N, 128))

    deg_st = _deg_call(dst_pad, ones128, zeros128).reshape(2, _N, 128)
    y1_st = _k1_call(x, W1, deg_st)
    a1_st = _seg_call(src2, dst_pad, y1_st.reshape(2 * _N, 128),
                      zeros128).reshape(2, _N, 128)
    y2_st = _k2_call(a1_st, y1_st, deg_st, b1r, W2)
    a2_st = _seg_call(src2, dst_pad, y2_st.reshape(2 * _N, 128),
                      zeros128).reshape(2, _N, 128)
    sums, cnts = _k3_call(a2_st, y2_st, deg_st, b2r, batch_r)
    out = _k4_call(sums, cnts, fcw_pad, fcb_r)
    return out[:, 0:1]


# speedup vs baseline: 10.0147x; 1.2280x over previous
import functools

import jax
import jax.numpy as jnp
from jax import lax
from jax.experimental import pallas as pl
from jax.experimental.pallas import tpu as pltpu
from jax.experimental.pallas import tpu_sc as plsc

_N = 10000
_E = 320000
_DIN = 128
_DH = 256
_G = 64

_NC = 2
_NS = 16
_CH = 128
_EPAD = 327680
_EPT = _EPAD // _NS
_EPT2 = _EPAD // (_NC * _NS)
_RPT = 624
_RLAST = _N - 15 * _RPT
_TRASH = _N
_NACC = _N + 8

_BLK = 1000

_f32 = jnp.float32


def _sc_mesh():
    return plsc.VectorSubcoreMesh(
        core_axis_name="c", subcore_axis_name="s",
        num_cores=_NC, num_subcores=_NS)


def _zero_acc(zeros_hbm, acc_sh, s):
    @pl.when(s < 15)
    def _():
        pltpu.sync_copy(zeros_hbm.at[pl.ds(0, _RPT)],
                        acc_sh.at[pl.ds(s * _RPT, _RPT)])

    @pl.when(s == 15)
    def _():
        pltpu.sync_copy(zeros_hbm, acc_sh.at[pl.ds(15 * _RPT, _RLAST)])


def _write_out(acc_sh, out_hbm, c, s):
    @pl.when(s < 15)
    def _():
        pltpu.sync_copy(acc_sh.at[pl.ds(s * _RPT, _RPT)],
                        out_hbm.at[pl.ds(c * _N + s * _RPT, _RPT)])

    @pl.when(s == 15)
    def _():
        pltpu.sync_copy(acc_sh.at[pl.ds(15 * _RPT, _RLAST)],
                        out_hbm.at[pl.ds(c * _N + 15 * _RPT, _RLAST)])


def _deg_body(dst_hbm, ones_hbm, zeros_hbm,
              deg_hbm,
              dst_v, ones_v, acc_sh, sem):
    c = lax.axis_index("c")
    s = lax.axis_index("s")
    _zero_acc(zeros_hbm, acc_sh, s)
    pltpu.sync_copy(ones_hbm, ones_v)
    plsc.subcore_barrier()

    base = (c * _NS + s) * _EPT2

    def step(g, carry):
        off = base + g * _CH
        pltpu.sync_copy(dst_hbm.at[pl.ds(off, _CH)], dst_v)
        pltpu.sync_copy(ones_v, acc_sh.at[dst_v], add=True)
        return carry

    lax.fori_loop(0, _EPT2 // _CH, step, 0)
    plsc.subcore_barrier()
    _write_out(acc_sh, deg_hbm, c, s)


def _deg_call(dst_pad, ones128, zeros128):
    fn = pl.kernel(
        _deg_body,
        out_type=jax.ShapeDtypeStruct((2 * _N, 128), _f32),
        mesh=_sc_mesh(),
        scratch_types=[
            pltpu.VMEM((_CH,), jnp.int32),
            pltpu.VMEM((_CH, 128), _f32),
            pltpu.VMEM_SHARED((_NACC, 128), _f32),
            pltpu.SemaphoreType.DMA,
        ],
    )
    return fn(dst_pad, ones128, zeros128)


def _seg_body(src2_hbm, dst_hbm, y_hbm, zeros_hbm,
              out_hbm,
              src_v, dst_v, rows_v, si, sg, ss, acc_sh):
    c = lax.axis_index("c")
    s = lax.axis_index("s")
    _zero_acc(zeros_hbm, acc_sh, s)
    plsc.subcore_barrier()

    n_chunks = _EPT // _CH
    base = c * _EPAD + s * _EPT
    dbase = s * _EPT

    def load_idx(g, ib, sem):
        pltpu.async_copy(src2_hbm.at[pl.ds(base + g * _CH, _CH)],
                         src_v[ib], sem)
        pltpu.async_copy(dst_hbm.at[pl.ds(dbase + g * _CH, _CH)],
                         dst_v[ib], sem)

    def drain_idx(ib, sem):
        pltpu.make_async_copy(src2_hbm.at[pl.ds(0, _CH)], src_v[ib], sem).wait()
        pltpu.make_async_copy(dst_hbm.at[pl.ds(0, _CH)], dst_v[ib], sem).wait()

    def drain_scatter(rb, sem):
        pltpu.make_async_copy(y_hbm.at[pl.ds(0, _CH)], rows_v[rb], sem).wait()

    pltpu.sync_copy(src2_hbm.at[pl.ds(base, _CH)], src_v[0])
    pltpu.sync_copy(dst_hbm.at[pl.ds(dbase, _CH)], dst_v[0])
    pltpu.sync_copy(src2_hbm.at[pl.ds(base + _CH, _CH)], src_v[1])
    pltpu.sync_copy(dst_hbm.at[pl.ds(dbase + _CH, _CH)], dst_v[1])

    def quad(p, carry):
        for b in range(4):
            g = 4 * p + b
            pb = b & 1

            @pl.when(g >= 2)
            def _():
                drain_scatter(pb, ss[pb])
                drain_idx(b, si[pb])

            @pl.when(g + 2 < n_chunks)
            def _():
                load_idx(g + 2, (b + 2) % 4, si[pb])

            pltpu.async_copy(y_hbm.at[src_v[b]], rows_v[pb], sg[pb])

            @pl.when(g >= 1)
            def _():
                pltpu.make_async_copy(y_hbm.at[pl.ds(0, _CH)],
                                      rows_v[1 - pb], sg[1 - pb]).wait()
                pltpu.async_copy(rows_v[1 - pb], acc_sh.at[dst_v[(b + 3) % 4]],
                                 ss[1 - pb], add=True)
        return carry

    lax.fori_loop(0, n_chunks // 4, quad, 0)

    pltpu.make_async_copy(y_hbm.at[pl.ds(0, _CH)], rows_v[1], sg[1]).wait()
    pltpu.async_copy(rows_v[1], acc_sh.at[dst_v[3]], ss[1], add=True)
    drain_scatter(0, ss[0])
    drain_scatter(1, ss[1])

    plsc.subcore_barrier()
    _write_out(acc_sh, out_hbm, c, s)


def _seg_call(src2, dst_pad, y_s, zeros128):
    fn = pl.kernel(
        _seg_body,
        out_type=jax.ShapeDtypeStruct((2 * _N, 128), _f32),
        mesh=_sc_mesh(),
        scratch_types=[
            [pltpu.VMEM((_CH,), jnp.int32) for _ in range(4)],
            [pltpu.VMEM((_CH,), jnp.int32) for _ in range(4)],
            [pltpu.VMEM((_CH, 128), _f32) for _ in range(2)],
            [pltpu.SemaphoreType.DMA for _ in range(2)],
            [pltpu.SemaphoreType.DMA for _ in range(2)],
            [pltpu.SemaphoreType.DMA for _ in range(2)],
            pltpu.VMEM_SHARED((_NACC, 128), _f32),
        ],
    )
    return fn(src2, dst_pad, y_s, zeros128)


def _dis_block(d_ref):
    deg = d_ref[...][0, :, 0:1] + d_ref[...][1, :, 0:1] + 1.0
    return lax.rsqrt(deg)


def _halves(st):
    return jnp.concatenate([st[0], st[1]], axis=1)


def _k1_body(x_ref, w1_ref, d_ref, y_ref):
    dis = _dis_block(d_ref)
    xw = jnp.dot(x_ref[...], w1_ref[...], preferred_element_type=_f32)
    y = xw * dis
    y_ref[...] = jnp.stack([y[:, :128], y[:, 128:]])


def _k1_call(x, W1, deg_st):
    grid = (_N // _BLK,)
    return pl.pallas_call(
        _k1_body,
        grid=grid,
        in_specs=[
            pl.BlockSpec((_BLK, _DIN), lambda i: (i, 0)),
            pl.BlockSpec((_DIN, _DH), lambda i: (0, 0)),
            pl.BlockSpec((2, _BLK, 128), lambda i: (0, i, 0)),
        ],
        out_specs=pl.BlockSpec((2, _BLK, 128), lambda i: (0, i, 0)),
        out_shape=jax.ShapeDtypeStruct((2, _N, 128), _f32),
    )(x, W1, deg_st)


def _k2_body(a_ref, y_ref, d_ref, b1_ref, w2_ref, o_ref):
    dis = _dis_block(d_ref)
    su = _halves(a_ref[...] + y_ref[...])
    h = jnp.maximum(su * dis + b1_ref[...][0:1, :], 0.0)
    y2 = jnp.dot(h, w2_ref[...], preferred_element_type=_f32) * dis
    o_ref[...] = jnp.stack([y2[:, :128], y2[:, 128:]])


def _k2_call(a_st, y_st, deg_st, b1r, W2):
    grid = (_N // _BLK,)
    st = pl.BlockSpec((2, _BLK, 128), lambda i: (0, i, 0))
    return pl.pallas_call(
        _k2_body,
        grid=grid,
        in_specs=[
            st, st,
            pl.BlockSpec((2, _BLK, 128), lambda i: (0, i, 0)),
            pl.BlockSpec((8, _DH), lambda i: (0, 0)),
            pl.BlockSpec((_DH, _DH), lambda i: (0, 0)),
        ],
        out_specs=st,
        out_shape=jax.ShapeDtypeStruct((2, _N, 128), _f32),
    )(a_st, y_st, deg_st, b1r, W2)


def _k3_body(a_ref, y_ref, d_ref, b2_ref, bat_ref, sums_ref, cnts_ref):
    i = pl.program_id(0)
    dis = _dis_block(d_ref)
    su = _halves(a_ref[...] + y_ref[...])
    h2 = su * dis + b2_ref[...][0:1, :]
    bb = bat_ref[...][:, 0:1]
    gio = lax.broadcasted_iota(jnp.int32, (_BLK, _G), 1)
    oh = (bb == gio).astype(_f32)
    psum = lax.dot_general(oh, h2, (((0,), (0,)), ((), ())),
                           preferred_element_type=_f32)
    pcnt = jnp.broadcast_to(jnp.sum(oh, axis=0)[:, None], (_G, 128))

    @pl.when(i == 0)
    def _():
        sums_ref[...] = psum
        cnts_ref[...] = pcnt

    @pl.when(i > 0)
    def _():
        sums_ref[...] = sums_ref[...] + psum
        cnts_ref[...] = cnts_ref[...] + pcnt


def _k3_call(a_st, y_st, deg_st, b2r, batch_r):
    grid = (_N // _BLK,)
    st = pl.BlockSpec((2, _BLK, 128), lambda i: (0, i, 0))
    return pl.pallas_call(
        _k3_body,
        grid=grid,
        in_specs=[
            st, st,
            pl.BlockSpec((2, _BLK, 128), lambda i: (0, i, 0)),
            pl.BlockSpec((8, _DH), lambda i: (0, 0)),
            pl.BlockSpec((_BLK, 128), lambda i: (i, 0)),
        ],
        out_specs=[
            pl.BlockSpec((_G, _DH), lambda i: (0, 0)),
            pl.BlockSpec((_G, 128), lambda i: (0, 0)),
        ],
        out_shape=[jax.ShapeDtypeStruct((_G, _DH), _f32),
                   jax.ShapeDtypeStruct((_G, 128), _f32)],
    )(a_st, y_st, deg_st, b2r, batch_r)


def _k4_body(sums_ref, cnts_ref, w_ref, b_ref, out_ref):
    pooled = sums_ref[...] / cnts_ref[...][:, 0:1]
    logits = jnp.dot(pooled, w_ref[...], preferred_element_type=_f32)
    out_ref[...] = jax.nn.sigmoid(logits + b_ref[...][0:1, 0:1])


def _k4_call(sums, cnts, fcw_pad, fcb_r):
    return pl.pallas_call(
        _k4_body,
        grid=(1,),
        in_specs=[
            pl.BlockSpec((_G, _DH), lambda i: (0, 0)),
            pl.BlockSpec((_G, 128), lambda i: (0, 0)),
            pl.BlockSpec((_DH, 128), lambda i: (0, 0)),
            pl.BlockSpec((8, 128), lambda i: (0, 0)),
        ],
        out_specs=pl.BlockSpec((_G, 128), lambda i: (0, 0)),
        out_shape=jax.ShapeDtypeStruct((_G, 128), _f32),
    )(sums, cnts, fcw_pad, fcb_r)


def kernel(x, edge_index, batch, W1, b1, W2, b2, fc_w, fc_b):
    npad = _EPAD - _E
    ei = edge_index.astype(jnp.int32)
    src_pad = jnp.concatenate([ei[0], jnp.zeros((npad,), jnp.int32)])
    dst_pad = jnp.concatenate([ei[1], jnp.full((npad,), _TRASH, jnp.int32)])
    src2 = jnp.concatenate([src_pad, src_pad + _N])

    ones128 = jnp.ones((_CH, 128), _f32)
    zeros128 = jnp.zeros((_RLAST, 128), _f32)

    b1r = jnp.broadcast_to(b1[None, :], (8, _DH))
    b2r = jnp.broadcast_to(b2[None, :], (8, _DH))
    fcw_pad = jnp.pad(fc_w, ((0, 0), (0, 127)))
    fcb_r = jnp.broadcast_to(fc_b[None, :], (8, 128))
    batch_r = jnp.broadcast_to(batch.astype(jnp.int32)[:, None], (_N, 128))

    deg_st = _deg_call(dst_pad, ones128, zeros128).reshape(2, _N, 128)
    y1_st = _k1_call(x, W1, deg_st)
    a1_st = _seg_call(src2, dst_pad, y1_st.reshape(2 * _N, 128),
                      zeros128).reshape(2, _N, 128)
    y2_st = _k2_call(a1_st, y1_st, deg_st, b1r, W2)
    a2_st = _seg_call(src2, dst_pad, y2_st.reshape(2 * _N, 128),
                      zeros128).reshape(2, _N, 128)
    sums, cnts = _k3_call(a2_st, y2_st, deg_st, b2r, batch_r)
    out = _k4_call(sums, cnts, fcw_pad, fcb_r)
    return out[:, 0:1]


# speedup vs baseline: 10.1495x; 1.0135x over previous
import functools

import jax
import jax.numpy as jnp
from jax import lax
from jax.experimental import pallas as pl
from jax.experimental.pallas import tpu as pltpu
from jax.experimental.pallas import tpu_sc as plsc

_N = 10000
_E = 320000
_DIN = 128
_DH = 256
_G = 64

_NC = 2
_NS = 16
_CH = 128
_CHS = 80
_EPAD = 327680
_EPT = _EPAD // _NS
_EPT2 = _EPAD // (_NC * _NS)
_RPT = 624
_RLAST = _N - 15 * _RPT
_TRASH = _N
_NACC = _N + 8

_BLK = 1000

_f32 = jnp.float32


def _sc_mesh():
    return plsc.VectorSubcoreMesh(
        core_axis_name="c", subcore_axis_name="s",
        num_cores=_NC, num_subcores=_NS)


def _zero_acc(zeros_hbm, acc_sh, s):
    @pl.when(s < 15)
    def _():
        pltpu.sync_copy(zeros_hbm.at[pl.ds(0, _RPT)],
                        acc_sh.at[pl.ds(s * _RPT, _RPT)])

    @pl.when(s == 15)
    def _():
        pltpu.sync_copy(zeros_hbm, acc_sh.at[pl.ds(15 * _RPT, _RLAST)])


def _write_out(acc_sh, out_hbm, c, s):
    @pl.when(s < 15)
    def _():
        pltpu.sync_copy(acc_sh.at[pl.ds(s * _RPT, _RPT)],
                        out_hbm.at[pl.ds(c * _N + s * _RPT, _RPT)])

    @pl.when(s == 15)
    def _():
        pltpu.sync_copy(acc_sh.at[pl.ds(15 * _RPT, _RLAST)],
                        out_hbm.at[pl.ds(c * _N + 15 * _RPT, _RLAST)])


def _deg_body(dst_hbm, ones_hbm, zeros_hbm,
              deg_hbm,
              dst_v, ones_v, acc_sh, sem):
    c = lax.axis_index("c")
    s = lax.axis_index("s")
    _zero_acc(zeros_hbm, acc_sh, s)
    pltpu.sync_copy(ones_hbm, ones_v)
    plsc.subcore_barrier()

    base = (c * _NS + s) * _EPT2

    def step(g, carry):
        off = base + g * _CH
        pltpu.sync_copy(dst_hbm.at[pl.ds(off, _CH)], dst_v)
        pltpu.sync_copy(ones_v, acc_sh.at[dst_v], add=True)
        return carry

    lax.fori_loop(0, _EPT2 // _CH, step, 0)
    plsc.subcore_barrier()
    _write_out(acc_sh, deg_hbm, c, s)


def _deg_call(dst_pad, ones128, zeros128):
    fn = pl.kernel(
        _deg_body,
        out_type=jax.ShapeDtypeStruct((2 * _N, 128), _f32),
        mesh=_sc_mesh(),
        scratch_types=[
            pltpu.VMEM((_CH,), jnp.int32),
            pltpu.VMEM((_CH, 128), _f32),
            pltpu.VMEM_SHARED((_NACC, 128), _f32),
            pltpu.SemaphoreType.DMA,
        ],
    )
    return fn(dst_pad, ones128, zeros128)


def _seg_body(src2_hbm, dst_hbm, y_hbm, zeros_hbm,
              out_hbm,
              src_v, dst_v, rows_v, si, sg, ss, acc_sh):
    c = lax.axis_index("c")
    s = lax.axis_index("s")
    _zero_acc(zeros_hbm, acc_sh, s)
    plsc.subcore_barrier()

    n_chunks = _EPT // _CHS
    base = c * _EPAD + s * _EPT
    dbase = s * _EPT

    def load_idx(g, ib, sem):
        pltpu.async_copy(src2_hbm.at[pl.ds(base + g * _CHS, _CHS)],
                         src_v[ib], sem)
        pltpu.async_copy(dst_hbm.at[pl.ds(dbase + g * _CHS, _CHS)],
                         dst_v[ib], sem)

    def drain_idx(ib, sem):
        pltpu.make_async_copy(src2_hbm.at[pl.ds(0, _CHS)], src_v[ib], sem).wait()
        pltpu.make_async_copy(dst_hbm.at[pl.ds(0, _CHS)], dst_v[ib], sem).wait()

    def drain_gather(rb):
        pltpu.make_async_copy(y_hbm.at[pl.ds(0, _CHS)], rows_v[rb], sg[rb]).wait()

    def drain_scatter(rb):
        pltpu.make_async_copy(y_hbm.at[pl.ds(0, _CHS)], rows_v[rb], ss[rb]).wait()

    def scatter(g_static_b, rb):
        pltpu.async_copy(rows_v[rb], acc_sh.at[dst_v[g_static_b]],
                         ss[rb], add=True)

    for g0 in range(4):
        pltpu.sync_copy(src2_hbm.at[pl.ds(base + g0 * _CHS, _CHS)], src_v[g0])
        pltpu.sync_copy(dst_hbm.at[pl.ds(dbase + g0 * _CHS, _CHS)], dst_v[g0])

    def octet(p, carry):
        for b in range(8):
            g = 8 * p + b
            rb = b % 4

            @pl.when(g >= 4)
            def _():
                drain_idx(b, si[rb])

            @pl.when(g + 4 < n_chunks)
            def _():
                load_idx(g + 4, (b + 4) % 8, si[rb])

            pltpu.async_copy(y_hbm.at[src_v[b]], rows_v[rb], sg[rb])

            @pl.when(g >= 2)
            def _():
                drain_gather((rb + 2) % 4)
                pltpu.sync_copy(rows_v[(rb + 2) % 4],
                                acc_sh.at[dst_v[(b + 6) % 8]], add=True)
        return carry

    lax.fori_loop(0, n_chunks // 8, octet, 0)

    drain_gather(2)
    pltpu.sync_copy(rows_v[2], acc_sh.at[dst_v[6]], add=True)
    drain_gather(3)
    pltpu.sync_copy(rows_v[3], acc_sh.at[dst_v[7]], add=True)

    plsc.subcore_barrier()
    _write_out(acc_sh, out_hbm, c, s)


def _seg_call(src2, dst_pad, y_s, zeros128):
    fn = pl.kernel(
        _seg_body,
        out_type=jax.ShapeDtypeStruct((2 * _N, 128), _f32),
        mesh=_sc_mesh(),
        scratch_types=[
            [pltpu.VMEM((_CHS,), jnp.int32) for _ in range(8)],
            [pltpu.VMEM((_CHS,), jnp.int32) for _ in range(8)],
            [pltpu.VMEM((_CHS, 128), _f32) for _ in range(4)],
            [pltpu.SemaphoreType.DMA for _ in range(4)],
            [pltpu.SemaphoreType.DMA for _ in range(4)],
            [pltpu.SemaphoreType.DMA for _ in range(4)],
            pltpu.VMEM_SHARED((_NACC, 128), _f32),
        ],
    )
    return fn(src2, dst_pad, y_s, zeros128)


def _dis_block(d_ref):
    deg = d_ref[...][0, :, 0:1] + d_ref[...][1, :, 0:1] + 1.0
    return lax.rsqrt(deg)


def _halves(st):
    return jnp.concatenate([st[0], st[1]], axis=1)


def _k1_body(x_ref, w1_ref, d_ref, y_ref):
    dis = _dis_block(d_ref)
    xw = jnp.dot(x_ref[...], w1_ref[...], preferred_element_type=_f32)
    y = xw * dis
    y_ref[...] = jnp.stack([y[:, :128], y[:, 128:]])


def _k1_call(x, W1, deg_st):
    grid = (_N // _BLK,)
    return pl.pallas_call(
        _k1_body,
        grid=grid,
        in_specs=[
            pl.BlockSpec((_BLK, _DIN), lambda i: (i, 0)),
            pl.BlockSpec((_DIN, _DH), lambda i: (0, 0)),
            pl.BlockSpec((2, _BLK, 128), lambda i: (0, i, 0)),
        ],
        out_specs=pl.BlockSpec((2, _BLK, 128), lambda i: (0, i, 0)),
        out_shape=jax.ShapeDtypeStruct((2, _N, 128), _f32),
    )(x, W1, deg_st)


def _k2_body(a_ref, y_ref, d_ref, b1_ref, w2_ref, o_ref):
    dis = _dis_block(d_ref)
    su = _halves(a_ref[...] + y_ref[...])
    h = jnp.maximum(su * dis + b1_ref[...][0:1, :], 0.0)
    y2 = jnp.dot(h, w2_ref[...], preferred_element_type=_f32) * dis
    o_ref[...] = jnp.stack([y2[:, :128], y2[:, 128:]])


def _k2_call(a_st, y_st, deg_st, b1r, W2):
    grid = (_N // _BLK,)
    st = pl.BlockSpec((2, _BLK, 128), lambda i: (0, i, 0))
    return pl.pallas_call(
        _k2_body,
        grid=grid,
        in_specs=[
            st, st,
            pl.BlockSpec((2, _BLK, 128), lambda i: (0, i, 0)),
            pl.BlockSpec((8, _DH), lambda i: (0, 0)),
            pl.BlockSpec((_DH, _DH), lambda i: (0, 0)),
        ],
        out_specs=st,
        out_shape=jax.ShapeDtypeStruct((2, _N, 128), _f32),
    )(a_st, y_st, deg_st, b1r, W2)


def _k3_body(a_ref, y_ref, d_ref, b2_ref, bat_ref, sums_ref, cnts_ref):
    i = pl.program_id(0)
    dis = _dis_block(d_ref)
    su = _halves(a_ref[...] + y_ref[...])
    h2 = su * dis + b2_ref[...][0:1, :]
    bb = bat_ref[...][:, 0:1]
    gio = lax.broadcasted_iota(jnp.int32, (_BLK, _G), 1)
    oh = (bb == gio).astype(_f32)
    psum = lax.dot_general(oh, h2, (((0,), (0,)), ((), ())),
                           preferred_element_type=_f32)
    pcnt = jnp.broadcast_to(jnp.sum(oh, axis=0)[:, None], (_G, 128))

    @pl.when(i == 0)
    def _():
        sums_ref[...] = psum
        cnts_ref[...] = pcnt

    @pl.when(i > 0)
    def _():
        sums_ref[...] = sums_ref[...] + psum
        cnts_ref[...] = cnts_ref[...] + pcnt


def _k3_call(a_st, y_st, deg_st, b2r, batch_r):
    grid = (_N // _BLK,)
    st = pl.BlockSpec((2, _BLK, 128), lambda i: (0, i, 0))
    return pl.pallas_call(
        _k3_body,
        grid=grid,
        in_specs=[
            st, st,
            pl.BlockSpec((2, _BLK, 128), lambda i: (0, i, 0)),
            pl.BlockSpec((8, _DH), lambda i: (0, 0)),
            pl.BlockSpec((_BLK, 128), lambda i: (i, 0)),
        ],
        out_specs=[
            pl.BlockSpec((_G, _DH), lambda i: (0, 0)),
            pl.BlockSpec((_G, 128), lambda i: (0, 0)),
        ],
        out_shape=[jax.ShapeDtypeStruct((_G, _DH), _f32),
                   jax.ShapeDtypeStruct((_G, 128), _f32)],
    )(a_st, y_st, deg_st, b2r, batch_r)


def _k4_body(sums_ref, cnts_ref, w_ref, b_ref, out_ref):
    pooled = sums_ref[...] / cnts_ref[...][:, 0:1]
    logits = jnp.dot(pooled, w_ref[...], preferred_element_type=_f32)
    out_ref[...] = jax.nn.sigmoid(logits + b_ref[...][0:1, 0:1])


def _k4_call(sums, cnts, fcw_pad, fcb_r):
    return pl.pallas_call(
        _k4_body,
        grid=(1,),
        in_specs=[
            pl.BlockSpec((_G, _DH), lambda i: (0, 0)),
            pl.BlockSpec((_G, 128), lambda i: (0, 0)),
            pl.BlockSpec((_DH, 128), lambda i: (0, 0)),
            pl.BlockSpec((8, 128), lambda i: (0, 0)),
        ],
        out_specs=pl.BlockSpec((_G, 128), lambda i: (0, 0)),
        out_shape=jax.ShapeDtypeStruct((_G, 128), _f32),
    )(sums, cnts, fcw_pad, fcb_r)


def kernel(x, edge_index, batch, W1, b1, W2, b2, fc_w, fc_b):
    npad = _EPAD - _E
    ei = edge_index.astype(jnp.int32)
    src_pad = jnp.concatenate([ei[0], jnp.zeros((npad,), jnp.int32)])
    dst_pad = jnp.concatenate([ei[1], jnp.full((npad,), _TRASH, jnp.int32)])
    src2 = jnp.concatenate([src_pad, src_pad + _N])

    ones128 = jnp.ones((_CH, 128), _f32)
    zeros128 = jnp.zeros((_RLAST, 128), _f32)

    b1r = jnp.broadcast_to(b1[None, :], (8, _DH))
    b2r = jnp.broadcast_to(b2[None, :], (8, _DH))
    fcw_pad = jnp.pad(fc_w, ((0, 0), (0, 127)))
    fcb_r = jnp.broadcast_to(fc_b[None, :], (8, 128))
    batch_r = jnp.broadcast_to(batch.astype(jnp.int32)[:, None], (_N, 128))

    deg_st = _deg_call(dst_pad, ones128, zeros128).reshape(2, _N, 128)
    y1_st = _k1_call(x, W1, deg_st)
    a1_st = _seg_call(src2, dst_pad, y1_st.reshape(2 * _N, 128),
                      zeros128).reshape(2, _N, 128)
    y2_st = _k2_call(a1_st, y1_st, deg_st, b1r, W2)
    a2_st = _seg_call(src2, dst_pad, y2_st.reshape(2 * _N, 128),
                      zeros128).reshape(2, _N, 128)
    sums, cnts = _k3_call(a2_st, y2_st, deg_st, b2r, batch_r)
    out = _k4_call(sums, cnts, fcw_pad, fcb_r)
    return out[:, 0:1]


# speedup vs baseline: 10.1709x; 1.0021x over previous
import functools

import jax
import jax.numpy as jnp
from jax import lax
from jax.experimental import pallas as pl
from jax.experimental.pallas import tpu as pltpu
from jax.experimental.pallas import tpu_sc as plsc

_N = 10000
_E = 320000
_DIN = 128
_DH = 256
_G = 64

_NC = 2
_NS = 16
_CH = 128
_CHS = 80
_EPAD = 327680
_EPT = _EPAD // _NS
_EPT2 = _EPAD // (_NC * _NS)
_RPT = 624
_RLAST = _N - 15 * _RPT
_TRASH = _N
_NACC = _N + 8

_BLK = 1000

_f32 = jnp.float32


def _sc_mesh():
    return plsc.VectorSubcoreMesh(
        core_axis_name="c", subcore_axis_name="s",
        num_cores=_NC, num_subcores=_NS)


def _zero_acc(zeros_hbm, acc_sh, s):
    @pl.when(s < 15)
    def _():
        pltpu.sync_copy(zeros_hbm.at[pl.ds(0, _RPT)],
                        acc_sh.at[pl.ds(s * _RPT, _RPT)])

    @pl.when(s == 15)
    def _():
        pltpu.sync_copy(zeros_hbm, acc_sh.at[pl.ds(15 * _RPT, _RLAST)])


def _write_out(acc_sh, out_hbm, c, s):
    @pl.when(s < 15)
    def _():
        pltpu.sync_copy(acc_sh.at[pl.ds(s * _RPT, _RPT)],
                        out_hbm.at[pl.ds(c * _N + s * _RPT, _RPT)])

    @pl.when(s == 15)
    def _():
        pltpu.sync_copy(acc_sh.at[pl.ds(15 * _RPT, _RLAST)],
                        out_hbm.at[pl.ds(c * _N + 15 * _RPT, _RLAST)])


def _deg_body(dst_hbm, ones_hbm, zeros_hbm,
              deg_hbm,
              dst_v, ones_v, acc_sh, sem):
    c = lax.axis_index("c")
    s = lax.axis_index("s")
    _zero_acc(zeros_hbm, acc_sh, s)
    pltpu.sync_copy(ones_hbm, ones_v)
    plsc.subcore_barrier()

    base = (c * _NS + s) * _EPT2

    def step(g, carry):
        off = base + g * _CH
        pltpu.sync_copy(dst_hbm.at[pl.ds(off, _CH)], dst_v)
        pltpu.sync_copy(ones_v, acc_sh.at[dst_v], add=True)
        return carry

    lax.fori_loop(0, _EPT2 // _CH, step, 0)
    plsc.subcore_barrier()
    _write_out(acc_sh, deg_hbm, c, s)


def _deg_call(dst_pad, ones128, zeros128):
    fn = pl.kernel(
        _deg_body,
        out_type=jax.ShapeDtypeStruct((2 * _N, 128), _f32),
        mesh=_sc_mesh(),
        scratch_types=[
            pltpu.VMEM((_CH,), jnp.int32),
            pltpu.VMEM((_CH, 128), _f32),
            pltpu.VMEM_SHARED((_NACC, 128), _f32),
            pltpu.SemaphoreType.DMA,
        ],
    )
    return fn(dst_pad, ones128, zeros128)


def _seg_body(src2_hbm, dst_hbm, y_hbm, zeros_hbm,
              out_hbm,
              src_v, dst_v, rows_v, si, sg, ss, acc_sh):
    c = lax.axis_index("c")
    s = lax.axis_index("s")
    _zero_acc(zeros_hbm, acc_sh, s)
    plsc.subcore_barrier()

    n_chunks = _EPT // _CHS
    base = c * _EPAD + s * _EPT
    dbase = s * _EPT

    def load_idx(g, ib, sem):
        pltpu.async_copy(src2_hbm.at[pl.ds(base + g * _CHS, _CHS)],
                         src_v[ib], sem)
        pltpu.async_copy(dst_hbm.at[pl.ds(dbase + g * _CHS, _CHS)],
                         dst_v[ib], sem)

    def drain_idx(ib, sem):
        pltpu.make_async_copy(src2_hbm.at[pl.ds(0, _CHS)], src_v[ib], sem).wait()
        pltpu.make_async_copy(dst_hbm.at[pl.ds(0, _CHS)], dst_v[ib], sem).wait()

    def drain_gather(rb):
        pltpu.make_async_copy(y_hbm.at[pl.ds(0, _CHS)], rows_v[rb], sg[rb]).wait()

    def drain_scatter(rb):
        pltpu.make_async_copy(y_hbm.at[pl.ds(0, _CHS)], rows_v[rb], ss[rb]).wait()

    def scatter(g_static_b, rb):
        pltpu.async_copy(rows_v[rb], acc_sh.at[dst_v[g_static_b]],
                         ss[rb], add=True)

    for g0 in range(4):
        pltpu.sync_copy(src2_hbm.at[pl.ds(base + g0 * _CHS, _CHS)], src_v[g0])
        pltpu.sync_copy(dst_hbm.at[pl.ds(dbase + g0 * _CHS, _CHS)], dst_v[g0])

    def octet(p, carry):
        for b in range(8):
            g = 8 * p + b
            rb = b % 4

            @pl.when(g >= 4)
            def _():
                drain_idx(b, si[rb])

            @pl.when(g + 4 < n_chunks)
            def _():
                load_idx(g + 4, (b + 4) % 8, si[rb])

            pltpu.async_copy(y_hbm.at[src_v[b]], rows_v[rb], sg[rb])

            @pl.when(g >= 3)
            def _():
                drain_gather((rb + 1) % 4)
                pltpu.sync_copy(rows_v[(rb + 1) % 4],
                                acc_sh.at[dst_v[(b + 5) % 8]], add=True)
        return carry

    lax.fori_loop(0, n_chunks // 8, octet, 0)

    drain_gather(1)
    pltpu.sync_copy(rows_v[1], acc_sh.at[dst_v[5]], add=True)
    drain_gather(2)
    pltpu.sync_copy(rows_v[2], acc_sh.at[dst_v[6]], add=True)
    drain_gather(3)
    pltpu.sync_copy(rows_v[3], acc_sh.at[dst_v[7]], add=True)

    plsc.subcore_barrier()
    _write_out(acc_sh, out_hbm, c, s)


def _seg_call(src2, dst_pad, y_s, zeros128):
    fn = pl.kernel(
        _seg_body,
        out_type=jax.ShapeDtypeStruct((2 * _N, 128), _f32),
        mesh=_sc_mesh(),
        scratch_types=[
            [pltpu.VMEM((_CHS,), jnp.int32) for _ in range(8)],
            [pltpu.VMEM((_CHS,), jnp.int32) for _ in range(8)],
            [pltpu.VMEM((_CHS, 128), _f32) for _ in range(4)],
            [pltpu.SemaphoreType.DMA for _ in range(4)],
            [pltpu.SemaphoreType.DMA for _ in range(4)],
            [pltpu.SemaphoreType.DMA for _ in range(4)],
            pltpu.VMEM_SHARED((_NACC, 128), _f32),
        ],
    )
    return fn(src2, dst_pad, y_s, zeros128)


def _dis_block(d_ref):
    deg = d_ref[...][0, :, 0:1] + d_ref[...][1, :, 0:1] + 1.0
    return lax.rsqrt(deg)


def _halves(st):
    return jnp.concatenate([st[0], st[1]], axis=1)


def _k1_body(x_ref, w1_ref, d_ref, y_ref):
    dis = _dis_block(d_ref)
    xw = jnp.dot(x_ref[...], w1_ref[...], preferred_element_type=_f32)
    y = xw * dis
    y_ref[...] = jnp.stack([y[:, :128], y[:, 128:]])


def _k1_call(x, W1, deg_st):
    grid = (_N // _BLK,)
    return pl.pallas_call(
        _k1_body,
        grid=grid,
        in_specs=[
            pl.BlockSpec((_BLK, _DIN), lambda i: (i, 0)),
            pl.BlockSpec((_DIN, _DH), lambda i: (0, 0)),
            pl.BlockSpec((2, _BLK, 128), lambda i: (0, i, 0)),
        ],
        out_specs=pl.BlockSpec((2, _BLK, 128), lambda i: (0, i, 0)),
        out_shape=jax.ShapeDtypeStruct((2, _N, 128), _f32),
    )(x, W1, deg_st)


def _k2_body(a_ref, y_ref, d_ref, b1_ref, w2_ref, o_ref):
    dis = _dis_block(d_ref)
    su = _halves(a_ref[...] + y_ref[...])
    h = jnp.maximum(su * dis + b1_ref[...][0:1, :], 0.0)
    y2 = jnp.dot(h, w2_ref[...], preferred_element_type=_f32) * dis
    o_ref[...] = jnp.stack([y2[:, :128], y2[:, 128:]])


def _k2_call(a_st, y_st, deg_st, b1r, W2):
    grid = (_N // _BLK,)
    st = pl.BlockSpec((2, _BLK, 128), lambda i: (0, i, 0))
    return pl.pallas_call(
        _k2_body,
        grid=grid,
        in_specs=[
            st, st,
            pl.BlockSpec((2, _BLK, 128), lambda i: (0, i, 0)),
            pl.BlockSpec((8, _DH), lambda i: (0, 0)),
            pl.BlockSpec((_DH, _DH), lambda i: (0, 0)),
        ],
        out_specs=st,
        out_shape=jax.ShapeDtypeStruct((2, _N, 128), _f32),
    )(a_st, y_st, deg_st, b1r, W2)


def _k3_body(a_ref, y_ref, d_ref, b2_ref, bat_ref, sums_ref, cnts_ref):
    i = pl.program_id(0)
    dis = _dis_block(d_ref)
    su = _halves(a_ref[...] + y_ref[...])
    h2 = su * dis + b2_ref[...][0:1, :]
    bb = bat_ref[...][:, 0:1]
    gio = lax.broadcasted_iota(jnp.int32, (_BLK, _G), 1)
    oh = (bb == gio).astype(_f32)
    psum = lax.dot_general(oh, h2, (((0,), (0,)), ((), ())),
                           preferred_element_type=_f32)
    pcnt = jnp.broadcast_to(jnp.sum(oh, axis=0)[:, None], (_G, 128))

    @pl.when(i == 0)
    def _():
        sums_ref[...] = psum
        cnts_ref[...] = pcnt

    @pl.when(i > 0)
    def _():
        sums_ref[...] = sums_ref[...] + psum
        cnts_ref[...] = cnts_ref[...] + pcnt


def _k3_call(a_st, y_st, deg_st, b2r, batch_r):
    grid = (_N // _BLK,)
    st = pl.BlockSpec((2, _BLK, 128), lambda i: (0, i, 0))
    return pl.pallas_call(
        _k3_body,
        grid=grid,
        in_specs=[
            st, st,
            pl.BlockSpec((2, _BLK, 128), lambda i: (0, i, 0)),
            pl.BlockSpec((8, _DH), lambda i: (0, 0)),
            pl.BlockSpec((_BLK, 128), lambda i: (i, 0)),
        ],
        out_specs=[
            pl.BlockSpec((_G, _DH), lambda i: (0, 0)),
            pl.BlockSpec((_G, 128), lambda i: (0, 0)),
        ],
        out_shape=[jax.ShapeDtypeStruct((_G, _DH), _f32),
                   jax.ShapeDtypeStruct((_G, 128), _f32)],
    )(a_st, y_st, deg_st, b2r, batch_r)


def _k4_body(sums_ref, cnts_ref, w_ref, b_ref, out_ref):
    pooled = sums_ref[...] / cnts_ref[...][:, 0:1]
    logits = jnp.dot(pooled, w_ref[...], preferred_element_type=_f32)
    out_ref[...] = jax.nn.sigmoid(logits + b_ref[...][0:1, 0:1])


def _k4_call(sums, cnts, fcw_pad, fcb_r):
    return pl.pallas_call(
        _k4_body,
        grid=(1,),
        in_specs=[
            pl.BlockSpec((_G, _DH), lambda i: (0, 0)),
            pl.BlockSpec((_G, 128), lambda i: (0, 0)),
            pl.BlockSpec((_DH, 128), lambda i: (0, 0)),
            pl.BlockSpec((8, 128), lambda i: (0, 0)),
        ],
        out_specs=pl.BlockSpec((_G, 128), lambda i: (0, 0)),
        out_shape=jax.ShapeDtypeStruct((_G, 128), _f32),
    )(sums, cnts, fcw_pad, fcb_r)


def kernel(x, edge_index, batch, W1, b1, W2, b2, fc_w, fc_b):
    npad = _EPAD - _E
    ei = edge_index.astype(jnp.int32)
    src_pad = jnp.concatenate([ei[0], jnp.zeros((npad,), jnp.int32)])
    dst_pad = jnp.concatenate([ei[1], jnp.full((npad,), _TRASH, jnp.int32)])
    src2 = jnp.concatenate([src_pad, src_pad + _N])

    ones128 = jnp.ones((_CH, 128), _f32)
    zeros128 = jnp.zeros((_RLAST, 128), _f32)

    b1r = jnp.broadcast_to(b1[None, :], (8, _DH))
    b2r = jnp.broadcast_to(b2[None, :], (8, _DH))
    fcw_pad = jnp.pad(fc_w, ((0, 0), (0, 127)))
    fcb_r = jnp.broadcast_to(fc_b[None, :], (8, 128))
    batch_r = jnp.broadcast_to(batch.astype(jnp.int32)[:, None], (_N, 128))

    deg_st = _deg_call(dst_pad, ones128, zeros128).reshape(2, _N, 128)
    y1_st = _k1_call(x, W1, deg_st)
    a1_st = _seg_call(src2, dst_pad, y1_st.reshape(2 * _N, 128),
                      zeros128).reshape(2, _N, 128)
    y2_st = _k2_call(a1_st, y1_st, deg_st, b1r, W2)
    a2_st = _seg_call(src2, dst_pad, y2_st.reshape(2 * _N, 128),
                      zeros128).reshape(2, _N, 128)
    sums, cnts = _k3_call(a2_st, y2_st, deg_st, b2r, batch_r)
    out = _k4_call(sums, cnts, fcw_pad, fcb_r)
    return out[:, 0:1]
